# baseline (speedup 1.0000x reference)
import functools

import jax
import jax.numpy as jnp
from jax.experimental import pallas as pl
from jax.experimental.pallas import tpu as pltpu

N = 10000
E = 160000


def _mm_body(a_ref, b_ref, o_ref):
    o_ref[...] = jnp.dot(a_ref[...], b_ref[...],
                         preferred_element_type=jnp.float32)


def _matmul(x, w, bm=512, bn=256):
    M, K = x.shape
    _, Nc = w.shape
    Mp = ((M + bm - 1) // bm) * bm
    Np = ((Nc + bn - 1) // bn) * bn
    if Mp != M:
        x = jnp.pad(x, ((0, Mp - M), (0, 0)))
    if Np != Nc:
        w = jnp.pad(w, ((0, 0), (0, Np - Nc)))
    out = pl.pallas_call(
        _mm_body,
        grid=(Mp // bm, Np // bn),
        in_specs=[
            pl.BlockSpec((bm, K), lambda i, j: (i, 0)),
            pl.BlockSpec((K, bn), lambda i, j: (0, j)),
        ],
        out_specs=pl.BlockSpec((bm, bn), lambda i, j: (i, j)),
        out_shape=jax.ShapeDtypeStruct((Mp, Np), jnp.float32),
        compiler_params=pltpu.CompilerParams(
            dimension_semantics=("parallel", "parallel")),
    )(x, w)
    return out[:M, :Nc]


def _edge_softmax(e, dst):
    m = jax.ops.segment_max(e, dst, num_segments=N)
    m = jnp.where(jnp.isfinite(m), m, 0.0)
    ex = jnp.exp(e - m[dst])
    s = jax.ops.segment_sum(ex, dst, num_segments=N)
    return ex / (s[dst] + 1e-16)


def _gat_layer(h, src, dst, W, al, ar, H, D, res, act):
    ft = _matmul(h, W).reshape(N, H, D)
    el = jnp.sum(ft * al[None, :, :], axis=-1)
    er = jnp.sum(ft * ar[None, :, :], axis=-1)
    e = jax.nn.leaky_relu(el[src] + er[dst], 0.2)
    a = _edge_softmax(e, dst)
    msg = ft[src] * a[:, :, None]
    out = jax.ops.segment_sum(msg, dst, num_segments=N)
    if isinstance(res, str) and res == 'id':
        out = out + h.reshape(N, H, D)
    elif res is not None:
        out = out + _matmul(h, res).reshape(N, H, D)
    if act:
        out = jax.nn.elu(out)
    return out


def kernel(inputs, W0, al0, ar0, W1, al1, ar1, W2, al2, ar2, res2, edge_index):
    src = edge_index[0]
    dst = edge_index[1]
    h = _gat_layer(inputs, src, dst, W0, al0, ar0, 4, 256, None, True)
    h = h.reshape(N, 1024)
    h = _gat_layer(h, src, dst, W1, al1, ar1, 4, 256, 'id', True)
    h = h.reshape(N, 1024)
    h = _gat_layer(h, src, dst, W2, al2, ar2, 6, 40, res2, False)
    return h.mean(axis=1)


# speedup vs baseline: 1.5232x; 1.5232x over previous
import functools

import jax
import jax.numpy as jnp
from jax import lax
from jax.experimental import pallas as pl
from jax.experimental.pallas import tpu as pltpu
from jax.experimental.pallas import tpu_sc as plsc

N = 10000
E = 160000
B = 400
NTILES = 16
EPT = E // NTILES
NB = EPT // B


def _mm_body(a_ref, b_ref, o_ref):
    o_ref[...] = jnp.dot(a_ref[...], b_ref[...],
                         preferred_element_type=jnp.float32)


def _matmul(x, w, bm=512, bn=256):
    M, K = x.shape
    _, Nc = w.shape
    Mp = ((M + bm - 1) // bm) * bm
    Npc = ((Nc + bn - 1) // bn) * bn
    if Mp != M:
        x = jnp.pad(x, ((0, Mp - M), (0, 0)))
    if Npc != Nc:
        w = jnp.pad(w, ((0, 0), (0, Npc - Nc)))
    out = pl.pallas_call(
        _mm_body,
        grid=(Mp // bm, Npc // bn),
        in_specs=[
            pl.BlockSpec((bm, K), lambda i, j: (i, 0)),
            pl.BlockSpec((K, bn), lambda i, j: (0, j)),
        ],
        out_specs=pl.BlockSpec((bm, bn), lambda i, j: (i, j)),
        out_shape=jax.ShapeDtypeStruct((Mp, Npc), jnp.float32),
        compiler_params=pltpu.CompilerParams(
            dimension_semantics=("parallel", "parallel")),
    )(x, w)
    return out[:M, :Nc]


def _edge_softmax_sc(elT, erT, src, dst, mv, H):
    H2 = H // 2
    H2N = ((H2 * N + 255) // 256) * 256
    STR = H2N // NTILES

    mesh = plsc.VectorSubcoreMesh(core_axis_name="c", subcore_axis_name="s")

    scratch = [
        pltpu.VMEM((H2 * N,), jnp.float32),
        pltpu.VMEM((H2 * N,), jnp.float32),
        pltpu.VMEM((H2N,), jnp.float32),
        pltpu.VMEM((STR,), jnp.float32),
        pltpu.VMEM((STR,), jnp.float32),
        pltpu.VMEM((B,), jnp.int32),
        pltpu.VMEM((B,), jnp.int32),
        pltpu.VMEM((H2 * B,), jnp.float32),
        pltpu.VMEM((16,), jnp.float32),
        pltpu.VMEM_SHARED((2 * H2N,), jnp.float32),
        pltpu.VMEM_SHARED((H2N,), jnp.float32),
    ]

    @functools.partial(
        pl.kernel, mesh=mesh,
        out_type=jax.ShapeDtypeStruct((H * E,), jnp.float32),
        compiler_params=pltpu.CompilerParams(needs_layout_passes=False),
        scratch_types=scratch)
    def body(el_h, er_h, src_h, dst_h, mv_h, out_h,
             el_v, er_v, s_v, acc_l, tmp_l, srcb, dstb, ab_v, mv_v,
             s_all, s_sh):
        c = lax.axis_index("c")
        t = lax.axis_index("s")
        ebase = t * EPT

        pltpu.sync_copy(el_h.at[pl.ds(c * (H2 * N), H2 * N)], el_v)
        pltpu.sync_copy(er_h.at[pl.ds(c * (H2 * N), H2 * N)], er_v)
        pltpu.sync_copy(mv_h.at[pl.ds(c * 16, 16)], mv_v)
        mvec = mv_v[...]
        zero16 = jnp.zeros((16,), jnp.float32)

        def zs(k, carry):
            s_v[pl.ds(k * 16, 16)] = zero16
            return carry
        lax.fori_loop(0, H2N // 16, zs, 0)

        def pa(b, carry):
            pltpu.sync_copy(src_h.at[pl.ds(ebase + b * B, B)], srcb)
            pltpu.sync_copy(dst_h.at[pl.ds(ebase + b * B, B)], dstb)

            def pag(gi, carry2):
                s16 = srcb[pl.ds(gi * 16, 16)]
                d16 = dstb[pl.ds(gi * 16, 16)]
                for h in range(H2):
                    el16 = plsc.load_gather(el_v, [s16 + h * N])
                    er16 = plsc.load_gather(er_v, [d16 + h * N])
                    e = el16 + er16
                    e = jnp.where(e > 0, e, 0.2 * e)
                    w = jnp.exp(e - mvec[h])
                    plsc.addupdate_scatter(s_v, [d16 + h * N], w)
                return carry2
            lax.fori_loop(0, B // 16, pag, 0)
            return carry
        lax.fori_loop(0, NB, pa, 0)

        def za(k, carry):
            acc_l[pl.ds(k * 16, 16)] = zero16
            return carry
        lax.fori_loop(0, STR // 16, za, 0)
        for r in range(8):
            @pl.when((t >= r * 2) & (t < r * 2 + 2))
            def _():
                pltpu.sync_copy(s_v, s_all.at[pl.ds((t - r * 2) * H2N,
                                                    H2N)])
            plsc.subcore_barrier()
            for q in range(2):
                pltpu.sync_copy(s_all.at[pl.ds(q * H2N + t * STR, STR)],
                                tmp_l)

                def ad(k, carry):
                    acc_l[pl.ds(k * 16, 16)] = (acc_l[pl.ds(k * 16, 16)]
                                                + tmp_l[pl.ds(k * 16, 16)])
                    return carry
                lax.fori_loop(0, STR // 16, ad, 0)
            plsc.subcore_barrier()
        pltpu.sync_copy(acc_l, s_sh.at[pl.ds(t * STR, STR)])
        plsc.subcore_barrier()
        pltpu.sync_copy(s_sh, s_v)

        def pc(b, carry):
            pltpu.sync_copy(src_h.at[pl.ds(ebase + b * B, B)], srcb)
            pltpu.sync_copy(dst_h.at[pl.ds(ebase + b * B, B)], dstb)

            def pcg(gi, carry2):
                s16 = srcb[pl.ds(gi * 16, 16)]
                d16 = dstb[pl.ds(gi * 16, 16)]
                for h in range(H2):
                    el16 = plsc.load_gather(el_v, [s16 + h * N])
                    er16 = plsc.load_gather(er_v, [d16 + h * N])
                    e = el16 + er16
                    e = jnp.where(e > 0, e, 0.2 * e)
                    w = jnp.exp(e - mvec[h])
                    sv = plsc.load_gather(s_v, [d16 + h * N])
                    ab_v[pl.ds(h * B + gi * 16, 16)] = w / (sv + 1e-16)
                return carry2
            lax.fori_loop(0, B // 16, pcg, 0)
            for h in range(H2):
                pltpu.sync_copy(
                    ab_v.at[pl.ds(h * B, B)],
                    out_h.at[pl.ds((c * H2 + h) * E + ebase + b * B, B)])
            return carry
        lax.fori_loop(0, NB, pc, 0)

    return body(elT, erT, src, dst, mv)


def _shifts(el, er, H):
    H2 = H // 2
    m = jnp.max(el, axis=0) + jnp.max(er, axis=0)
    z = jnp.zeros((16 - H2,), jnp.float32)
    return jnp.concatenate([m[:H2], z, m[H2:], z])


def _edge_weights(el, er, src, dst, H):
    mv = _shifts(el, er, H)
    a = _edge_softmax_sc(el.T.reshape(-1), er.T.reshape(-1),
                         src, dst, mv, H)
    return a.reshape(H, E).T


def kernel(inputs, W0, al0, ar0, W1, al1, ar1, W2, al2, ar2, res2,
           edge_index):
    src = edge_index[0]
    dst = edge_index[1]

    wal0 = jnp.einsum('khd,hd->kh', W0.reshape(256, 4, 256), al0)
    war0 = jnp.einsum('khd,hd->kh', W0.reshape(256, 4, 256), ar0)
    wal1 = jnp.einsum('khd,hd->kh', W1.reshape(1024, 4, 256), al1)
    war1 = jnp.einsum('khd,hd->kh', W1.reshape(1024, 4, 256), ar1)
    wal2 = jnp.einsum('khd,hd->kh', W2.reshape(1024, 6, 40), al2)
    war2 = jnp.einsum('khd,hd->kh', W2.reshape(1024, 6, 40), ar2)

    lr0 = _matmul(inputs, jnp.concatenate([wal0, war0], axis=1))
    el0, er0 = lr0[:, :4], lr0[:, 4:8]
    a0 = _edge_weights(el0, er0, src, dst, 4)
    xg = inputs[src]
    aggs = [jax.ops.segment_sum(xg * a0[:, h:h + 1], dst, num_segments=N)
            for h in range(4)]
    outs = [_matmul(aggs[h], W0[:, h * 256:(h + 1) * 256])
            for h in range(4)]
    h1 = jax.nn.elu(jnp.concatenate(outs, axis=1))

    ft1c = _matmul(h1, jnp.concatenate([W1, wal1, war1], axis=1))
    ft1, el1, er1 = ft1c[:, :1024], ft1c[:, 1024:1028], ft1c[:, 1028:1032]
    a1 = _edge_weights(el1, er1, src, dst, 4)
    msg1 = ft1[src].reshape(E, 4, 256) * a1[:, :, None]
    agg1 = jax.ops.segment_sum(msg1, dst, num_segments=N).reshape(N, 1024)
    h2 = jax.nn.elu(agg1 + h1)

    ft2c = _matmul(h2, jnp.concatenate([W2, res2, wal2, war2], axis=1))
    ft2, resv = ft2c[:, :240], ft2c[:, 240:480]
    el2, er2 = ft2c[:, 480:486], ft2c[:, 486:492]
    a2 = _edge_weights(el2, er2, src, dst, 6)
    msg2 = ft2[src].reshape(E, 6, 40) * a2[:, :, None]
    agg2 = jax.ops.segment_sum(msg2, dst, num_segments=N)
    out = agg2 + resv.reshape(N, 6, 40)
    return out.mean(axis=1)
